# Initial kernel scaffold; baseline (speedup 1.0000x reference)
import jax
import jax.numpy as jnp
from jax.experimental import pallas as pl


def kernel(x, edge_index, Wl1, bl1, Wr1, g1, b1, Wl2, bl2, Wr2, g2, b2):
    raise NotImplementedError("write your pallas kernel here")


# speedup vs baseline: 6.8824x; 6.8824x over previous
import functools

import jax
import jax.numpy as jnp
from jax import lax
from jax.experimental import pallas as pl
from jax.experimental.pallas import tpu as pltpu
from jax.experimental.pallas import tpu_sc as plsc

N = 10000
E = 320000
D = 128

NC = 2
NS = 16
NW = NC * NS
EPW = E // NW
C = 128
FULL = EPW // C
TAIL = EPW - FULL * C
NPAD = 10240
RPT = NPAD // NS
CPT = NPAD // NS

_mesh = plsc.VectorSubcoreMesh(core_axis_name="c", subcore_axis_name="s")


def _make_sc_agg(with_cnt: bool):
  out_type = [jax.ShapeDtypeStruct((NC, NPAD, D), jnp.float32)]
  if with_cnt:
    out_type.append(jax.ShapeDtypeStruct((NC, NPAD), jnp.float32))
  scratch = [
      pltpu.VMEM((C,), jnp.int32),
      pltpu.VMEM((C,), jnp.int32),
      pltpu.VMEM((C, D), jnp.float32),
      pltpu.VMEM((TAIL,), jnp.int32),
      pltpu.VMEM((TAIL,), jnp.int32),
      pltpu.VMEM((TAIL, D), jnp.float32),
      pltpu.VMEM_SHARED((NPAD, D), jnp.float32),
      pltpu.SemaphoreType.DMA,
  ]
  if with_cnt:
    scratch += [
        pltpu.VMEM((C,), jnp.float32),
        pltpu.VMEM((TAIL,), jnp.float32),
        pltpu.VMEM((CPT,), jnp.float32),
        pltpu.VMEM_SHARED((NPAD,), jnp.float32),
    ]

  def body(src_hbm, dst_hbm, x_hbm, zrows_hbm, *rest):
    if with_cnt:
      (zcnt_hbm, p_hbm, cnt_hbm, sidx, didx, rows, sidx_t, didx_t, rows_t,
       acc, sem, ones, ones_t, cbuf, cacc) = rest
    else:
      (p_hbm, sidx, didx, rows, sidx_t, didx_t, rows_t, acc, sem) = rest
    c = lax.axis_index("c")
    s = lax.axis_index("s")
    wid = c * NS + s
    base = wid * EPW

    pltpu.sync_copy(zrows_hbm, rows)
    row0 = s * RPT
    for k in range(RPT // C):
      pltpu.sync_copy(rows, acc.at[pl.ds(row0 + k * C, C)])
    if with_cnt:
      pltpu.sync_copy(zcnt_hbm.at[pl.ds(0, CPT)], cbuf)
      pltpu.sync_copy(cbuf, cacc.at[pl.ds(s * CPT, CPT)])
      for j in range(C // 16):
        ones[pl.ds(j * 16, 16)] = jnp.ones((16,), jnp.float32)
      ones_t[...] = jnp.ones((TAIL,), jnp.float32)
    plsc.subcore_barrier()

    def chunk(k, carry):
      off = base + k * C
      pltpu.sync_copy(src_hbm.at[pl.ds(off, C)], sidx)
      pltpu.sync_copy(dst_hbm.at[pl.ds(off, C)], didx)
      pltpu.async_copy(x_hbm.at[sidx], rows, sem).wait()
      pltpu.sync_copy(rows, acc.at[didx], add=True)
      if with_cnt:
        pltpu.sync_copy(ones, cacc.at[didx], add=True)
      return carry

    lax.fori_loop(0, FULL, chunk, 0)

    off = base + FULL * C
    pltpu.sync_copy(src_hbm.at[pl.ds(off, TAIL)], sidx_t)
    pltpu.sync_copy(dst_hbm.at[pl.ds(off, TAIL)], didx_t)
    pltpu.async_copy(x_hbm.at[sidx_t], rows_t, sem).wait()
    pltpu.sync_copy(rows_t, acc.at[didx_t], add=True)
    if with_cnt:
      pltpu.sync_copy(ones_t, cacc.at[didx_t], add=True)

    plsc.subcore_barrier()

    for k in range(RPT // C):
      r = row0 + k * C
      pltpu.sync_copy(acc.at[pl.ds(r, C)], rows)
      pltpu.sync_copy(rows, p_hbm.at[c, pl.ds(r, C)])
    if with_cnt:
      pltpu.sync_copy(cacc.at[pl.ds(s * CPT, CPT)], cbuf)
      pltpu.sync_copy(cbuf, cnt_hbm.at[c, pl.ds(s * CPT, CPT)])

  return pl.kernel(body, out_type=out_type, mesh=_mesh,
                   scratch_types=scratch)


_sc_agg_cnt = _make_sc_agg(True)
_sc_agg = _make_sc_agg(False)


def _dense(p, inv, xin, wlt, wrt, bl, g, b, relu):
  def body(p_ref, inv_ref, x_ref, wl_ref, wr_ref, bl_ref, g_ref, b_ref, o_ref):
    agg = p_ref[0, :N] + p_ref[1, :N]
    mean = agg * inv_ref[...]
    t = (jnp.dot(mean, wl_ref[...], preferred_element_type=jnp.float32)
         + jnp.dot(x_ref[...], wr_ref[...], preferred_element_type=jnp.float32)
         + bl_ref[...])
    mu = jnp.mean(t, axis=0, keepdims=True)
    var = jnp.mean((t - mu) ** 2, axis=0, keepdims=True)
    h = (t - mu) * lax.rsqrt(var + 1e-5) * g_ref[...] + b_ref[...]
    if relu:
      h = jnp.maximum(h, 0.0)
    o_ref[...] = h

  return pl.pallas_call(
      body, out_shape=jax.ShapeDtypeStruct((N, D), jnp.float32),
  )(p, inv, xin, wlt, wrt, bl, g, b)


def kernel(x, edge_index, Wl1, bl1, Wr1, g1, b1, Wl2, bl2, Wr2, g2, b2):
  src = edge_index[0]
  dst = edge_index[1]
  zrows = jnp.zeros((C, D), jnp.float32)
  zcnt = jnp.zeros((NPAD,), jnp.float32)

  p1, cnt = _sc_agg_cnt(src, dst, x, zrows, zcnt)
  cnt_tot = cnt[0, :N] + cnt[1, :N]
  inv = (1.0 / jnp.maximum(cnt_tot, 1.0))[:, None]

  h = _dense(p1, inv, x, Wl1.T, Wr1.T, bl1[None, :], g1[None, :], b1[None, :],
             relu=True)
  (p2,) = _sc_agg(src, dst, h, zrows)
  out = _dense(p2, inv, h, Wl2.T, Wr2.T, bl2[None, :], g2[None, :],
               b2[None, :], relu=False)
  return out


# speedup vs baseline: 12.3947x; 1.8009x over previous
import functools

import jax
import jax.numpy as jnp
from jax import lax
from jax.experimental import pallas as pl
from jax.experimental.pallas import tpu as pltpu
from jax.experimental.pallas import tpu_sc as plsc

N = 10000
E = 320000
D = 128

NC = 2
NS = 16
NW = NC * NS
EPW = E // NW
C = 128
FULL = EPW // C
TAIL = EPW - FULL * C
NPAD = 10240
RPT = NPAD // NS
CPT = NPAD // NS

_mesh = plsc.VectorSubcoreMesh(core_axis_name="c", subcore_axis_name="s")


def _make_sc_agg(with_cnt: bool):
  out_type = [jax.ShapeDtypeStruct((NC, NPAD, D), jnp.float32)]
  if with_cnt:
    out_type.append(jax.ShapeDtypeStruct((NC, NPAD), jnp.float32))
  scratch = [
      pltpu.VMEM((C,), jnp.int32),
      pltpu.VMEM((C,), jnp.int32),
      pltpu.VMEM((C, D), jnp.float32),
      pltpu.VMEM((C,), jnp.int32),
      pltpu.VMEM((C,), jnp.int32),
      pltpu.VMEM((C, D), jnp.float32),
      pltpu.VMEM((TAIL,), jnp.int32),
      pltpu.VMEM((TAIL,), jnp.int32),
      pltpu.VMEM((TAIL, D), jnp.float32),
      pltpu.VMEM_SHARED((NPAD, D), jnp.float32),
      pltpu.SemaphoreType.DMA,
      pltpu.SemaphoreType.DMA,
      pltpu.SemaphoreType.DMA,
      pltpu.SemaphoreType.DMA,
  ]
  if with_cnt:
    scratch += [
        pltpu.VMEM((C,), jnp.float32),
        pltpu.VMEM((TAIL,), jnp.float32),
        pltpu.VMEM((CPT,), jnp.float32),
        pltpu.VMEM_SHARED((NPAD,), jnp.float32),
    ]

  def body(src_hbm, dst_hbm, x_hbm, zrows_hbm, *rest):
    if with_cnt:
      (zcnt_hbm, p_hbm, cnt_hbm, sidx0, didx0, rows0, sidx1, didx1, rows1,
       sidx_t, didx_t, rows_t, acc, isem0, isem1, gsem0, gsem1,
       ones, ones_t, cbuf, cacc) = rest
    else:
      (p_hbm, sidx0, didx0, rows0, sidx1, didx1, rows1,
       sidx_t, didx_t, rows_t, acc, isem0, isem1, gsem0, gsem1) = rest
    c = lax.axis_index("c")
    s = lax.axis_index("s")
    wid = c * NS + s
    base = wid * EPW
    bufs = ((sidx0, didx0, rows0, isem0, gsem0),
            (sidx1, didx1, rows1, isem1, gsem1))

    def idx_start(off, b):
      pltpu.async_copy(src_hbm.at[pl.ds(off, C)], b[0], b[3])
      pltpu.async_copy(dst_hbm.at[pl.ds(off, C)], b[1], b[3])

    def idx_wait(off, b):
      pltpu.make_async_copy(src_hbm.at[pl.ds(off, C)], b[0], b[3]).wait()
      pltpu.make_async_copy(dst_hbm.at[pl.ds(off, C)], b[1], b[3]).wait()

    def gather_start(b):
      pltpu.async_copy(x_hbm.at[b[0]], b[2], b[4])

    def gather_wait(b):
      pltpu.make_async_copy(x_hbm.at[b[0]], b[2], b[4]).wait()

    def scatter(b):
      pltpu.sync_copy(b[2], acc.at[b[1]], add=True)
      if with_cnt:
        pltpu.sync_copy(ones, cacc.at[b[1]], add=True)

    pltpu.sync_copy(zrows_hbm, rows0)
    row0 = s * RPT
    for k in range(RPT // C):
      pltpu.sync_copy(rows0, acc.at[pl.ds(row0 + k * C, C)])
    if with_cnt:
      pltpu.sync_copy(zcnt_hbm.at[pl.ds(0, CPT)], cbuf)
      pltpu.sync_copy(cbuf, cacc.at[pl.ds(s * CPT, CPT)])
      for j in range(C // 16):
        ones[pl.ds(j * 16, 16)] = jnp.ones((16,), jnp.float32)
      ones_t[...] = jnp.ones((TAIL,), jnp.float32)
    plsc.subcore_barrier()

    idx_start(base, bufs[0])
    idx_wait(base, bufs[0])
    gather_start(bufs[0])
    idx_start(base + C, bufs[1])

    def pair(i, carry):
      for j in range(2):
        k = 2 * i + j
        cur, nxt = bufs[j], bufs[1 - j]
        idx_wait(base + (k + 1) * C, nxt)
        gather_start(nxt)
        gather_wait(cur)
        scatter(cur)
        idx_start(base + (k + 2) * C, cur)
      return carry

    lax.fori_loop(0, (FULL - 2) // 2, pair, 0)

    off_t = base + FULL * C
    idx_wait(base + (FULL - 1) * C, bufs[1])
    gather_start(bufs[1])
    gather_wait(bufs[0])
    scatter(bufs[0])
    pltpu.async_copy(src_hbm.at[pl.ds(off_t, TAIL)], sidx_t, isem0)
    pltpu.async_copy(dst_hbm.at[pl.ds(off_t, TAIL)], didx_t, isem0)

    pltpu.make_async_copy(src_hbm.at[pl.ds(off_t, TAIL)], sidx_t, isem0).wait()
    pltpu.make_async_copy(dst_hbm.at[pl.ds(off_t, TAIL)], didx_t, isem0).wait()
    pltpu.async_copy(x_hbm.at[sidx_t], rows_t, gsem0)
    gather_wait(bufs[1])
    scatter(bufs[1])

    pltpu.make_async_copy(x_hbm.at[sidx_t], rows_t, gsem0).wait()
    pltpu.sync_copy(rows_t, acc.at[didx_t], add=True)
    if with_cnt:
      pltpu.sync_copy(ones_t, cacc.at[didx_t], add=True)

    plsc.subcore_barrier()

    for k in range(RPT // C):
      r = row0 + k * C
      pltpu.sync_copy(acc.at[pl.ds(r, C)], rows0)
      pltpu.sync_copy(rows0, p_hbm.at[c, pl.ds(r, C)])
    if with_cnt:
      pltpu.sync_copy(cacc.at[pl.ds(s * CPT, CPT)], cbuf)
      pltpu.sync_copy(cbuf, cnt_hbm.at[c, pl.ds(s * CPT, CPT)])

  return pl.kernel(body, out_type=out_type, mesh=_mesh,
                   scratch_types=scratch)


_sc_agg_cnt = _make_sc_agg(True)
_sc_agg = _make_sc_agg(False)


def _dense(p, inv, xin, wlt, wrt, bl, g, b, relu):
  def body(p_ref, inv_ref, x_ref, wl_ref, wr_ref, bl_ref, g_ref, b_ref, o_ref):
    agg = p_ref[0, :N] + p_ref[1, :N]
    mean = agg * inv_ref[...]
    t = (jnp.dot(mean, wl_ref[...], preferred_element_type=jnp.float32)
         + jnp.dot(x_ref[...], wr_ref[...], preferred_element_type=jnp.float32)
         + bl_ref[...])
    mu = jnp.mean(t, axis=0, keepdims=True)
    var = jnp.mean((t - mu) ** 2, axis=0, keepdims=True)
    h = (t - mu) * lax.rsqrt(var + 1e-5) * g_ref[...] + b_ref[...]
    if relu:
      h = jnp.maximum(h, 0.0)
    o_ref[...] = h

  return pl.pallas_call(
      body, out_shape=jax.ShapeDtypeStruct((N, D), jnp.float32),
  )(p, inv, xin, wlt, wrt, bl, g, b)


def kernel(x, edge_index, Wl1, bl1, Wr1, g1, b1, Wl2, bl2, Wr2, g2, b2):
  src = edge_index[0]
  dst = edge_index[1]
  zrows = jnp.zeros((C, D), jnp.float32)
  zcnt = jnp.zeros((NPAD,), jnp.float32)

  p1, cnt = _sc_agg_cnt(src, dst, x, zrows, zcnt)
  cnt_tot = cnt[0, :N] + cnt[1, :N]
  inv = (1.0 / jnp.maximum(cnt_tot, 1.0))[:, None]

  h = _dense(p1, inv, x, Wl1.T, Wr1.T, bl1[None, :], g1[None, :], b1[None, :],
             relu=True)
  (p2,) = _sc_agg(src, dst, h, zrows)
  out = _dense(p2, inv, h, Wl2.T, Wr2.T, bl2[None, :], g2[None, :],
               b2[None, :], relu=False)
  return out
